# Initial kernel scaffold; baseline (speedup 1.0000x reference)
import jax
import jax.numpy as jnp
from jax.experimental import pallas as pl


def kernel(embeds_u_1, embeds_u_2, embeds_u_3, embeds_u_4, embeds_v, v_embed, w_ur1_W, w_ur1_b, w_ur2_W, w_ur2_b, w_vr1_W, w_vr1_b, w_vr2_W, w_vr2_b, w_uv1_W, w_uv1_b, w_uv2_W, w_uv2_b, w_uv3_W, w_uv3_b, bn1_g, bn1_b, bn2_g, bn2_b, bn3_g, bn3_b, bn4_g, bn4_b, nodes_v, hist):
    raise NotImplementedError("write your pallas kernel here")


# speedup vs baseline: 13.2158x; 13.2158x over previous
import functools

import jax
import jax.numpy as jnp
import numpy as np
from jax import lax
from jax.experimental import pallas as pl
from jax.experimental.pallas import tpu as pltpu
from jax.experimental.pallas import tpu_sc as plsc

B = 4096
H = 200
V = 100000
D = 128

NC = 2
NS = 16
NW = NC * NS
QPW = B // NW
LANES = 16
HA = 104
HB = H - HA
BIG = np.float32(3.0e38)


def _knn_body(nodes_hbm, hist_hbm, table_hbm, out_hbm,
              nodes_v, hist_v, new_v, rows_v, dmin_v, sem0, sem1):
    wid = lax.axis_index("s") * NC + lax.axis_index("c")
    base = wid * QPW

    pltpu.sync_copy(nodes_hbm.at[pl.ds(base, QPW)], nodes_v)
    pltpu.sync_copy(hist_hbm.at[pl.ds(base * H, QPW * H)], hist_v)
    pltpu.async_copy(table_hbm.at[nodes_v], new_v, sem0).wait()

    sems = (sem0, sem1)

    def issue(q, slot):
        sem = sems[slot]
        qoff = pl.multiple_of(q * H, 8)
        pltpu.async_copy(table_hbm.at[hist_v.at[pl.ds(qoff, HA)]],
                         rows_v.at[slot, pl.ds(0, HA), :], sem)
        pltpu.async_copy(table_hbm.at[hist_v.at[pl.ds(qoff + HA, HB)]],
                         rows_v.at[slot, pl.ds(HA, HB), :], sem)

    def drain(slot):
        pltpu.make_async_copy(table_hbm.at[pl.ds(0, H), :],
                              rows_v.at[slot], sems[slot]).wait()

    lanemask = lax.iota(jnp.int32, LANES) == (LANES - 1)

    def compute(q, slot):
        n = [new_v[q, pl.ds(j * LANES, LANES)] for j in range(D // LANES)]

        def hbody(h, dmin):
            s = None
            for j in range(D // LANES):
                o = rows_v[slot, h, pl.ds(j * LANES, LANES)]
                df = n[j] - o
                sq = df * df
                s = sq if s is None else s + sq
            cs = jnp.cumsum(s)
            dv = jnp.where(lanemask, cs, BIG)
            return jnp.minimum(dmin, dv)

        dmin = lax.fori_loop(0, H, hbody, jnp.full((LANES,), BIG, jnp.float32))
        dmin_v[pl.ds(pl.multiple_of(q * LANES, 8), LANES)] = dmin

    issue(0, 0)

    def qbody(i, carry):
        q0 = 2 * i
        issue(q0 + 1, 1)
        drain(0)
        compute(q0, 0)

        @pl.when(q0 + 2 < QPW)
        def _():
            issue(q0 + 2, 0)

        drain(1)
        compute(q0 + 1, 1)
        return carry

    lax.fori_loop(0, QPW // 2, qbody, 0)
    pltpu.sync_copy(dmin_v, out_hbm.at[pl.ds(base * LANES, QPW * LANES)])


@functools.cache
def _make_knn():
    return pl.kernel(
        _knn_body,
        out_type=jax.ShapeDtypeStruct((B * LANES,), jnp.float32),
        mesh=plsc.VectorSubcoreMesh(core_axis_name="c", subcore_axis_name="s",
                                    num_cores=NC, num_subcores=NS),
        compiler_params=pltpu.CompilerParams(needs_layout_passes=False),
        scratch_types=[
            pltpu.VMEM((QPW,), jnp.int32),
            pltpu.VMEM((QPW * H,), jnp.int32),
            pltpu.VMEM((QPW, D), jnp.float32),
            pltpu.VMEM((2, H, D), jnp.float32),
            pltpu.VMEM((QPW * LANES,), jnp.float32),
            pltpu.SemaphoreType.DMA,
            pltpu.SemaphoreType.DMA,
        ],
    )


def _head_body(u1_ref, u2_ref, u3_ref, u4_ref, ev_ref, dslab_ref,
               wur1_ref, bur1_ref, wur2_ref, bur2_ref,
               wvr1_ref, bvr1_ref, wvr2_ref, bvr2_ref,
               wuv1a_ref, wuv1b_ref, buv1_ref, wuv2_ref, buv2_ref,
               wuv3_ref, buv3_ref,
               bn1g_ref, bn1b_ref, bn2g_ref, bn2b_ref,
               bn3g_ref, bn3b_ref, bn4g_ref, bn4b_ref,
               out_ref):
    u1 = u1_ref[...]
    u2 = u2_ref[...]
    u3 = u3_ref[...]
    u4 = u4_ref[...]

    def rnorm(a, b):
        dlt = a - b
        return jnp.sqrt(jnp.sum(dlt * dlt, axis=1, keepdims=True))

    c_u = (rnorm(u1, u2) + rnorm(u2, u3) + rnorm(u3, u4)) * jnp.float32(1.0 / 3.0)

    s0 = float(np.exp(-4.0) + np.exp(-3.0) + np.exp(-2.0) + np.exp(-1.0))
    fused = (u1 * jnp.float32(np.exp(-4.0) / s0)
             + u2 * jnp.float32(np.exp(-3.0) / s0)
             + u3 * jnp.float32(np.exp(-2.0) / s0)
             + u4 * jnp.float32(np.exp(-1.0) / s0))

    def bn(x, g_ref, b_ref):
        m = jnp.mean(x, axis=0, keepdims=True)
        xc = x - m
        v = jnp.mean(xc * xc, axis=0, keepdims=True)
        return xc / jnp.sqrt(v + 1e-5) * g_ref[...] + b_ref[...]

    def mm(x, w_ref):
        return jnp.dot(x, w_ref[...], preferred_element_type=jnp.float32)

    x_u = jnp.maximum(bn(mm(fused, wur1_ref) + bur1_ref[...], bn1g_ref, bn1b_ref), 0.0)
    x_u = mm(x_u, wur2_ref) + bur2_ref[...]
    x_v = jnp.maximum(bn(mm(ev_ref[...], wvr1_ref) + bvr1_ref[...], bn2g_ref, bn2b_ref), 0.0)
    x_v = mm(x_v, wvr2_ref) + bvr2_ref[...]
    x = jnp.maximum(bn(mm(x_u, wuv1a_ref) + mm(x_v, wuv1b_ref) + buv1_ref[...],
                       bn3g_ref, bn3b_ref), 0.0)
    x = jnp.maximum(bn(mm(x, wuv2_ref) + buv2_ref[...], bn4g_ref, bn4b_ref), 0.0)
    scores = mm(x, wuv3_ref) + buv3_ref[...]

    d_min = jnp.sqrt(jnp.min(dslab_ref[...], axis=1, keepdims=True))
    dlo = jnp.min(d_min)
    dhi = jnp.max(d_min)
    tmp = (d_min - dlo) / (dhi - dlo)
    unexp = jnp.float32(6.0) * tmp * jnp.exp(jnp.float32(-6.0) * tmp)
    clo = jnp.min(c_u)
    chi = jnp.max(c_u)
    c_n = (c_u - clo) / (chi - clo)
    out_ref[...] = scores + unexp * c_n


_head = pl.pallas_call(
    _head_body,
    out_shape=jax.ShapeDtypeStruct((B, 1), jnp.float32),
)


def kernel(embeds_u_1, embeds_u_2, embeds_u_3, embeds_u_4, embeds_v, v_embed,
           w_ur1_W, w_ur1_b, w_ur2_W, w_ur2_b, w_vr1_W, w_vr1_b, w_vr2_W, w_vr2_b,
           w_uv1_W, w_uv1_b, w_uv2_W, w_uv2_b, w_uv3_W, w_uv3_b,
           bn1_g, bn1_b, bn2_g, bn2_b, bn3_g, bn3_b, bn4_g, bn4_b,
           nodes_v, hist):
    dslab = _make_knn()(nodes_v.astype(jnp.int32),
                        hist.astype(jnp.int32).reshape(-1),
                        v_embed).reshape(B, LANES)
    r2 = lambda a: a.reshape(1, -1)
    ratings = _head(
        embeds_u_1, embeds_u_2, embeds_u_3, embeds_u_4, embeds_v, dslab,
        w_ur1_W, r2(w_ur1_b), w_ur2_W, r2(w_ur2_b),
        w_vr1_W, r2(w_vr1_b), w_vr2_W, r2(w_vr2_b),
        w_uv1_W[:D], w_uv1_W[D:], r2(w_uv1_b), w_uv2_W, r2(w_uv2_b),
        w_uv3_W, r2(w_uv3_b),
        r2(bn1_g), r2(bn1_b), r2(bn2_g), r2(bn2_b),
        r2(bn3_g), r2(bn3_b), r2(bn4_g), r2(bn4_b),
    )
    return jnp.squeeze(ratings)
